# baseline (speedup 1.0000x reference)
import jax
import jax.numpy as jnp
from jax.experimental import pallas as pl


BLOCK_S = 1024


def _add_kernel(x_ref, t_ref, o_ref):
    o_ref[...] = x_ref[...] + t_ref[...][None, :, :]


def kernel(x, table):
    batch, seq_len, d_model = x.shape
    grid = (seq_len // BLOCK_S, batch)
    return pl.pallas_call(
        _add_kernel,
        grid=grid,
        in_specs=[
            pl.BlockSpec((1, BLOCK_S, d_model), lambda i, b: (b, i, 0)),
            pl.BlockSpec((BLOCK_S, d_model), lambda i, b: (i, 0)),
        ],
        out_specs=pl.BlockSpec((1, BLOCK_S, d_model), lambda i, b: (b, i, 0)),
        out_shape=jax.ShapeDtypeStruct((batch, seq_len, d_model), x.dtype),
    )(x, table[:seq_len])


# speedup vs baseline: 1.0653x; 1.0653x over previous
import jax
import jax.numpy as jnp
from jax.experimental import pallas as pl


BLOCK_S = 2048


def _add_kernel(x_ref, t_ref, o_ref):
    o_ref[...] = x_ref[...] + t_ref[...][None, :, :]


def kernel(x, table):
    batch, seq_len, d_model = x.shape
    grid = (seq_len // BLOCK_S, batch)
    return pl.pallas_call(
        _add_kernel,
        grid=grid,
        in_specs=[
            pl.BlockSpec((1, BLOCK_S, d_model), lambda i, b: (b, i, 0)),
            pl.BlockSpec((BLOCK_S, d_model), lambda i, b: (i, 0)),
        ],
        out_specs=pl.BlockSpec((1, BLOCK_S, d_model), lambda i, b: (b, i, 0)),
        out_shape=jax.ShapeDtypeStruct((batch, seq_len, d_model), x.dtype),
    )(x, table[:seq_len])
